# baseline (speedup 1.0000x reference)
import functools

import jax
import jax.numpy as jnp
from jax import lax
from jax.experimental import pallas as pl
from jax.experimental.pallas import tpu as pltpu
from jax.experimental.pallas import tpu_sc as plsc

_NC = 2
_NS = 16
_NW = _NC * _NS
_CH = 128


def _sc_gather(table, idx):
    n, d = table.shape
    e = idx.shape[0]
    per_w = e // _NW
    n_iter = per_w // _CH
    mesh = plsc.VectorSubcoreMesh(core_axis_name="c", subcore_axis_name="s")

    @functools.partial(
        pl.kernel,
        out_type=jax.ShapeDtypeStruct((e, d), jnp.float32),
        mesh=mesh,
        scratch_types=[
            pltpu.VMEM((_CH,), jnp.int32),
            pltpu.VMEM((_CH, d), jnp.float32),
            pltpu.SemaphoreType.DMA,
        ],
    )
    def gather_kernel(table_hbm, idx_hbm, out_hbm, idx_v, rows_v, sem):
        wid = lax.axis_index("s") * _NC + lax.axis_index("c")
        base0 = wid * per_w

        def body(i, carry):
            base = base0 + i * _CH
            pltpu.sync_copy(idx_hbm.at[pl.ds(base, _CH)], idx_v)
            pltpu.async_copy(table_hbm.at[idx_v], rows_v, sem).wait()
            pltpu.sync_copy(rows_v, out_hbm.at[pl.ds(base, _CH)])
            return carry

        lax.fori_loop(0, n_iter, body, 0)

    return gather_kernel(table, idx)


def _sc_scatter_add(msgs, idx, zeros_nd):
    e, d = msgs.shape
    n = zeros_nd.shape[0]
    per_w = e // _NW
    n_iter = per_w // _CH
    rows_per_sub = (n // _NS) // 8 * 8
    tail_base = _NS * rows_per_sub
    tail_rows = n - tail_base
    mesh = plsc.VectorSubcoreMesh(core_axis_name="c", subcore_axis_name="s")

    @functools.partial(
        pl.kernel,
        out_type=jax.ShapeDtypeStruct((_NC * n, d), jnp.float32),
        mesh=mesh,
        scratch_types=[
            pltpu.VMEM((_CH,), jnp.int32),
            pltpu.VMEM((_CH, d), jnp.float32),
            pltpu.VMEM_SHARED((n, d), jnp.float32),
            pltpu.SemaphoreType.DMA,
        ],
    )
    def scatter_kernel(msgs_hbm, idx_hbm, zeros_hbm, out_hbm, idx_v, rows_v, acc_sh, sem):
        cid = lax.axis_index("c")
        sid = lax.axis_index("s")
        wid = sid * _NC + cid
        r0 = sid * rows_per_sub
        pltpu.sync_copy(zeros_hbm.at[pl.ds(r0, rows_per_sub)],
                        acc_sh.at[pl.ds(r0, rows_per_sub)])
        if tail_rows:
            @pl.when(sid == 0)
            def _zero_tail():
                pltpu.sync_copy(zeros_hbm.at[pl.ds(tail_base, tail_rows)],
                                acc_sh.at[pl.ds(tail_base, tail_rows)])
        plsc.subcore_barrier()

        base0 = wid * per_w

        def body(i, carry):
            base = base0 + i * _CH
            pltpu.sync_copy(idx_hbm.at[pl.ds(base, _CH)], idx_v)
            pltpu.sync_copy(msgs_hbm.at[pl.ds(base, _CH)], rows_v)
            pltpu.sync_copy(rows_v, acc_sh.at[idx_v], add=True)
            return carry

        lax.fori_loop(0, n_iter, body, 0)
        plsc.subcore_barrier()
        pltpu.sync_copy(acc_sh.at[pl.ds(r0, rows_per_sub)],
                        out_hbm.at[pl.ds(cid * n + r0, rows_per_sub)])
        if tail_rows:
            @pl.when(sid == 0)
            def _dump_tail():
                pltpu.sync_copy(acc_sh.at[pl.ds(tail_base, tail_rows)],
                                out_hbm.at[pl.ds(cid * n + tail_base, tail_rows)])

    return scatter_kernel(msgs, idx, zeros_nd)


_SELU_ALPHA = 1.6732632423543772
_SELU_SCALE = 1.0507009873554805


def _tc_edge_math(gathered, et, w_cat, b_cat):
    e, d = gathered.shape
    l_types = et.shape[1]
    o = w_cat.shape[-1]
    k = (o - d) // 2
    eb = 512
    grid = e // eb

    def body(se_ref, et_ref, w_ref, b_ref, out_ref):
        se = se_ref[...]
        ets = et_ref[...]
        acc = jnp.dot(ets, b_ref[...], preferred_element_type=jnp.float32)
        for l in range(l_types):
            x = (se * ets[:, l][:, None]).astype(jnp.bfloat16)
            acc = acc + jnp.dot(x, w_ref[l],
                                preferred_element_type=jnp.float32)
        p_src = acc[:, :k]
        p_qry = acc[:, k:2 * k]
        p_msg = acc[:, 2 * k:]
        s = _SELU_SCALE * jnp.where(p_src > 0, p_src,
                                    _SELU_ALPHA * (jnp.exp(p_src) - 1.0))
        scores = jnp.sum(s * p_qry, axis=1, keepdims=True)
        iew = jnp.mean(ets, axis=1, keepdims=True)
        out_ref[...] = (iew * jnp.exp(scores)) * p_msg

    return pl.pallas_call(
        body,
        grid=(grid,),
        in_specs=[
            pl.BlockSpec((eb, d), lambda i: (i, 0)),
            pl.BlockSpec((eb, l_types), lambda i: (i, 0)),
            pl.BlockSpec((l_types, d, o), lambda i: (0, 0, 0)),
            pl.BlockSpec((l_types, o), lambda i: (0, 0)),
        ],
        out_specs=pl.BlockSpec((eb, d), lambda i: (i, 0)),
        out_shape=jax.ShapeDtypeStruct((e, d), jnp.float32),
    )(gathered, et, w_cat, b_cat)


def _tc_norm_accum(partials, out_prev):
    n, d = out_prev.shape
    nb = 1000
    grid = n // nb

    def body(p_ref, prev_ref, cur_ref, out_ref):
        new = p_ref[0] + p_ref[1]
        ssq = jnp.sum(new * new, axis=1, keepdims=True)
        nrm = new * lax.rsqrt(jnp.maximum(ssq, 1e-12))
        cur_ref[...] = nrm
        out_ref[...] = prev_ref[...] + nrm

    return pl.pallas_call(
        body,
        grid=(grid,),
        in_specs=[
            pl.BlockSpec((2, nb, d), lambda i: (0, i, 0)),
            pl.BlockSpec((nb, d), lambda i: (i, 0)),
        ],
        out_specs=[
            pl.BlockSpec((nb, d), lambda i: (i, 0)),
            pl.BlockSpec((nb, d), lambda i: (i, 0)),
        ],
        out_shape=[
            jax.ShapeDtypeStruct((n, d), jnp.float32),
            jax.ShapeDtypeStruct((n, d), jnp.float32),
        ],
    )(partials, out_prev)


def kernel(entity_embeds, sources, targets, edge_types, W_src, b_src,
           W_query, W_msg, b_msg, W_ni, b_ni, W_ie):
    n, d = entity_embeds.shape
    c, s, l_types = edge_types.shape
    e = c * s
    src = sources.reshape(e)
    tgt = targets.reshape(e)
    et = edge_types.reshape(e, l_types)
    w_cat = jnp.concatenate([W_src, W_query, W_msg], axis=-1).astype(jnp.bfloat16)
    b_cat = jnp.concatenate([b_src, jnp.zeros_like(b_src), b_msg], axis=-1)
    zeros_nd = jnp.zeros((n, d), jnp.float32)

    out = entity_embeds
    cur = entity_embeds
    for _ in range(3):
        gathered = _sc_gather(cur, src)
        msgs = _tc_edge_math(gathered, et, w_cat, b_cat)
        partials = _sc_scatter_add(msgs, tgt, zeros_nd).reshape(2, n, d)
        cur, out = _tc_norm_accum(partials, out)
    return out


# speedup vs baseline: 1.4220x; 1.4220x over previous
import functools

import jax
import jax.numpy as jnp
from jax import lax
from jax.experimental import pallas as pl
from jax.experimental.pallas import tpu as pltpu
from jax.experimental.pallas import tpu_sc as plsc

_NC = 2
_NS = 16
_NW = _NC * _NS
_CH = 128


def _sc_gather(table, idx):
    n, d = table.shape
    e = idx.shape[0]
    per_w = e // _NW
    n_iter = per_w // _CH
    mesh = plsc.VectorSubcoreMesh(core_axis_name="c", subcore_axis_name="s")

    @functools.partial(
        pl.kernel,
        out_type=jax.ShapeDtypeStruct((e, d), jnp.float32),
        mesh=mesh,
        scratch_types=[
            pltpu.VMEM((_CH,), jnp.int32),
            pltpu.VMEM((_CH, d), jnp.float32),
            pltpu.SemaphoreType.DMA,
        ],
    )
    def gather_kernel(table_hbm, idx_hbm, out_hbm, idx_v, rows_v, sem):
        wid = lax.axis_index("s") * _NC + lax.axis_index("c")
        base0 = wid * per_w

        def body(i, carry):
            base = base0 + i * _CH
            pltpu.sync_copy(idx_hbm.at[pl.ds(base, _CH)], idx_v)
            pltpu.async_copy(table_hbm.at[idx_v], rows_v, sem).wait()
            pltpu.sync_copy(rows_v, out_hbm.at[pl.ds(base, _CH)])
            return carry

        lax.fori_loop(0, n_iter, body, 0)

    return gather_kernel(table, idx)


def _sc_scatter_add(msgs, idx, zeros_nd):
    e, d = msgs.shape
    n = zeros_nd.shape[0]
    per_w = e // _NW
    n_iter = per_w // _CH
    rows_per_sub = (n // _NS) // 8 * 8
    tail_base = _NS * rows_per_sub
    tail_rows = n - tail_base
    mesh = plsc.VectorSubcoreMesh(core_axis_name="c", subcore_axis_name="s")

    @functools.partial(
        pl.kernel,
        out_type=jax.ShapeDtypeStruct((_NC * n, d), jnp.float32),
        mesh=mesh,
        scratch_types=[
            pltpu.VMEM((_CH,), jnp.int32),
            pltpu.VMEM((_CH, d), jnp.float32),
            pltpu.VMEM_SHARED((n, d), jnp.float32),
            pltpu.SemaphoreType.DMA,
        ],
    )
    def scatter_kernel(msgs_hbm, idx_hbm, zeros_hbm, out_hbm, idx_v, rows_v, acc_sh, sem):
        cid = lax.axis_index("c")
        sid = lax.axis_index("s")
        wid = sid * _NC + cid
        r0 = sid * rows_per_sub
        pltpu.sync_copy(zeros_hbm.at[pl.ds(r0, rows_per_sub)],
                        acc_sh.at[pl.ds(r0, rows_per_sub)])
        if tail_rows:
            @pl.when(sid == 0)
            def _zero_tail():
                pltpu.sync_copy(zeros_hbm.at[pl.ds(tail_base, tail_rows)],
                                acc_sh.at[pl.ds(tail_base, tail_rows)])
        plsc.subcore_barrier()

        base0 = wid * per_w

        def body(i, carry):
            base = base0 + i * _CH
            pltpu.sync_copy(idx_hbm.at[pl.ds(base, _CH)], idx_v)
            pltpu.sync_copy(msgs_hbm.at[pl.ds(base, _CH)], rows_v)
            pltpu.sync_copy(rows_v, acc_sh.at[idx_v], add=True)
            return carry

        lax.fori_loop(0, n_iter, body, 0)
        plsc.subcore_barrier()
        pltpu.sync_copy(acc_sh.at[pl.ds(r0, rows_per_sub)],
                        out_hbm.at[pl.ds(cid * n + r0, rows_per_sub)])
        if tail_rows:
            @pl.when(sid == 0)
            def _dump_tail():
                pltpu.sync_copy(acc_sh.at[pl.ds(tail_base, tail_rows)],
                                out_hbm.at[pl.ds(cid * n + tail_base, tail_rows)])

    return scatter_kernel(msgs, idx, zeros_nd)


_SELU_ALPHA = 1.6732632423543772
_SELU_SCALE = 1.0507009873554805


def _tc_edge_math(gathered, et, w_flat, b_cat, e_mat):
    e, d = gathered.shape
    l_types = et.shape[1]
    o = w_flat.shape[-1]
    k = (o - d) // 2
    eb = 1024
    grid = e // eb

    def body(se_ref, et_ref, w_ref, b_ref, emat_ref, out_ref):
        se = se_ref[...]
        ets = et_ref[...]
        et_wide = jnp.broadcast_to(ets.astype(jnp.bfloat16)[:, :, None],
                                   (eb, l_types, d)).reshape(eb, l_types * d)
        se_wide = jnp.tile(se.astype(jnp.bfloat16), (1, l_types))
        u = et_wide * se_wide
        acc = jnp.dot(u, w_ref[...], preferred_element_type=jnp.float32)
        acc = acc + jnp.dot(ets, b_ref[...], preferred_element_type=jnp.float32)
        p_src = acc[:, :k]
        p_qry = acc[:, k:2 * k]
        p_msg = acc[:, 2 * k:]
        s = _SELU_SCALE * jnp.where(p_src > 0, p_src,
                                    _SELU_ALPHA * (jnp.exp(p_src) - 1.0))
        scores = jnp.sum(s * p_qry, axis=1, keepdims=True)
        iew = jnp.mean(ets, axis=1, keepdims=True)
        out_ref[...] = (iew * jnp.exp(scores)) * p_msg

    return pl.pallas_call(
        body,
        grid=(grid,),
        in_specs=[
            pl.BlockSpec((eb, d), lambda i: (i, 0)),
            pl.BlockSpec((eb, l_types), lambda i: (i, 0)),
            pl.BlockSpec((l_types * d, o), lambda i: (0, 0)),
            pl.BlockSpec((l_types, o), lambda i: (0, 0)),
            pl.BlockSpec((l_types, l_types * d), lambda i: (0, 0)),
        ],
        out_specs=pl.BlockSpec((eb, d), lambda i: (i, 0)),
        out_shape=jax.ShapeDtypeStruct((e, d), jnp.float32),
    )(gathered, et, w_flat, b_cat, e_mat)


def _tc_norm_accum(partials, out_prev):
    n, d = out_prev.shape
    nb = 1000
    grid = n // nb

    def body(p_ref, prev_ref, cur_ref, out_ref):
        new = p_ref[0] + p_ref[1]
        ssq = jnp.sum(new * new, axis=1, keepdims=True)
        nrm = new * lax.rsqrt(jnp.maximum(ssq, 1e-12))
        cur_ref[...] = nrm
        out_ref[...] = prev_ref[...] + nrm

    return pl.pallas_call(
        body,
        grid=(grid,),
        in_specs=[
            pl.BlockSpec((2, nb, d), lambda i: (0, i, 0)),
            pl.BlockSpec((nb, d), lambda i: (i, 0)),
        ],
        out_specs=[
            pl.BlockSpec((nb, d), lambda i: (i, 0)),
            pl.BlockSpec((nb, d), lambda i: (i, 0)),
        ],
        out_shape=[
            jax.ShapeDtypeStruct((n, d), jnp.float32),
            jax.ShapeDtypeStruct((n, d), jnp.float32),
        ],
    )(partials, out_prev)


def kernel(entity_embeds, sources, targets, edge_types, W_src, b_src,
           W_query, W_msg, b_msg, W_ni, b_ni, W_ie):
    n, d = entity_embeds.shape
    c, s, l_types = edge_types.shape
    e = c * s
    src = sources.reshape(e)
    tgt = targets.reshape(e)
    et = edge_types.reshape(e, l_types)
    w_cat = jnp.concatenate([W_src, W_query, W_msg], axis=-1)
    o = w_cat.shape[-1]
    w_flat = w_cat.reshape(l_types * d, o).astype(jnp.bfloat16)
    b_cat = jnp.concatenate([b_src, jnp.zeros_like(b_src), b_msg], axis=-1)
    e_mat = jnp.kron(jnp.eye(l_types, dtype=jnp.float32),
                     jnp.ones((1, d), jnp.float32)).astype(jnp.bfloat16)
    zeros_nd = jnp.zeros((n, d), jnp.float32)

    out = entity_embeds
    cur = entity_embeds
    for _ in range(3):
        gathered = _sc_gather(cur, src)
        msgs = _tc_edge_math(gathered, et, w_flat, b_cat, e_mat)
        partials = _sc_scatter_add(msgs, tgt, zeros_nd).reshape(2, n, d)
        cur, out = _tc_norm_accum(partials, out)
    return out
